# baseline (speedup 1.0000x reference)
import jax
import jax.numpy as jnp
from jax import lax
from jax.experimental import pallas as pl
from jax.experimental.pallas import tpu as pltpu
from jax.experimental.pallas import tpu_sc as plsc

VOCAB = 1000000
EMBED_DIM = 32
BATCH = 4096
HIST = 200

NC = 2
NS = 16
NW = NC * NS

LANE = 128
H8 = HIST // 8
CB = BATCH // LANE
TILES = H8 * CB
TPW = TILES // NW
CPW = TPW * 2


def _body(xr, table_hbm, out_hbm,
          idxs0, idxs1, rows0, rows1, dst0, dst1,
          is0, is1, gs0, gs1, ss0, ss1):
    idx_s = (idxs0, idxs1)
    rows_v = (rows0, rows1)
    dst_v = (dst0, dst1)
    i_s = (is0, is1)
    gat_s = (gs0, gs1)
    st_s = (ss0, ss1)

    wid = lax.axis_index("s") * NC + lax.axis_index("c")
    iota = lax.iota(jnp.int32, 16)

    def tile_qq(g):
        t = TPW * wid + g // 2
        return t, lax.rem(g, 2)

    def start_idx(g, b):
        t, qq = tile_qq(g)
        pltpu.async_copy(xr.at[t, pl.ds(4 * qq, 4)], idx_s[b], i_s[b])

    def wait_idx(b):
        pltpu.make_async_copy(
            xr.at[0, pl.ds(0, 4)], idx_s[b], i_s[b]
        ).wait()

    def fire(b):
        for d in range(4):
            pltpu.async_copy(
                table_hbm.at[idx_s[b].at[d]],
                rows_v[b].at[pl.ds(LANE * d, LANE)],
                gat_s[b],
            )

    def drain_gat(b):
        pltpu.make_async_copy(
            table_hbm.at[pl.ds(0, 4 * LANE)],
            rows_v[b],
            gat_s[b],
        ).wait()

    def out_slice(g):
        t, qq = tile_qq(g)
        h8 = t // CB
        c = lax.rem(t, CB)
        return out_hbm.at[pl.ds(32 * h8 + 16 * qq, 16), c]

    def extract(b):
        rows = rows_v[b]
        dst = dst_v[b]
        for d in range(4):

            def jbody(j, carry):
                rowv = 128 * d + 16 * j + iota

                def ebody(e4, c2):
                    colb = jnp.full((16,), 8 * e4, jnp.int32)
                    for s in range(8):
                        v = plsc.load_gather(rows, [rowv, colb + s])
                        dst[4 * d + e4, s, pl.ds(16 * j, 16)] = v
                    return c2

                lax.fori_loop(0, 4, ebody, 0)
                return carry

            lax.fori_loop(0, 8, jbody, 0)

    def start_store(g, b):
        pltpu.async_copy(dst_v[b], out_slice(g), st_s[b])

    def wait_store(g, b):
        pltpu.make_async_copy(dst_v[b], out_slice(g), st_s[b]).wait()

    def chunk(g, k, fire_next, wait_st, idx2):
        b = k % 2
        nb = (k + 1) % 2
        if fire_next:
            wait_idx(nb)
        drain_gat(b)
        if fire_next:
            fire(nb)
        if wait_st:
            wait_store(g - 2, b)
        extract(b)
        start_store(g, b)
        if idx2:
            start_idx(g + 2, b)

    start_idx(0, 0)
    start_idx(1, 1)
    wait_idx(0)
    fire(0)
    chunk(0, 0, True, False, True)
    chunk(1, 1, True, False, True)

    def step(it, carry):
        g = 2 + 2 * it
        chunk(g, 0, True, True, True)
        chunk(g + 1, 1, True, True, True)
        return carry

    lax.fori_loop(0, (CPW - 4) // 2, step, 0)

    chunk(CPW - 2, 0, True, True, False)
    chunk(CPW - 1, 1, False, True, False)
    wait_store(CPW - 2, 0)
    wait_store(CPW - 1, 1)


@jax.jit
def _embed(xr, table):
    mesh = plsc.VectorSubcoreMesh(core_axis_name="c", subcore_axis_name="s")
    fn = pl.kernel(
        _body,
        out_type=jax.ShapeDtypeStruct((4 * HIST, CB, 8, LANE), jnp.float32),
        mesh=mesh,
        scratch_types=[
            pltpu.VMEM((4, LANE), jnp.int32),
            pltpu.VMEM((4, LANE), jnp.int32),
            pltpu.VMEM((4 * LANE, EMBED_DIM), jnp.float32),
            pltpu.VMEM((4 * LANE, EMBED_DIM), jnp.float32),
            pltpu.VMEM((16, 8, LANE), jnp.float32),
            pltpu.VMEM((16, 8, LANE), jnp.float32),
            pltpu.SemaphoreType.DMA,
            pltpu.SemaphoreType.DMA,
            pltpu.SemaphoreType.DMA,
            pltpu.SemaphoreType.DMA,
            pltpu.SemaphoreType.DMA,
            pltpu.SemaphoreType.DMA,
        ],
        compiler_params=pltpu.CompilerParams(
            use_tc_tiling_on_sc=False, needs_layout_passes=False
        ),
    )
    return fn(xr, table)


def kernel(x, table):
    xi = x.astype(jnp.int32)
    xr = (
        xi.T.reshape(H8, 8, CB, LANE)
        .transpose(0, 2, 1, 3)
        .reshape(TILES, 8, LANE)
    )
    out5 = _embed(xr, table)
    out = (
        out5.reshape(HIST, 4, CB, 8, LANE)
        .transpose(2, 4, 0, 1, 3)
        .reshape(BATCH, HIST, EMBED_DIM)
    )
    return out


# speedup vs baseline: 1.2988x; 1.2988x over previous
import jax
import jax.numpy as jnp
from jax import lax
from jax.experimental import pallas as pl
from jax.experimental.pallas import tpu as pltpu
from jax.experimental.pallas import tpu_sc as plsc

VOCAB = 1000000
EMBED_DIM = 32
BATCH = 4096
HIST = 200

NC = 2
NS = 16
NW = NC * NS

LANE = 128
H8 = HIST // 8
CB = BATCH // LANE
TILES = H8 * CB
TPW = TILES // NW
CPW = TPW * 2


def _body(xr, table_hbm, out_hbm,
          idxs0, idxs1, rows0, rows1, dst0, dst1,
          is0, is1, gs0, gs1, ss0, ss1):
    idx_s = (idxs0, idxs1)
    rows_v = (rows0, rows1)
    dst_v = (dst0, dst1)
    i_s = (is0, is1)
    gat_s = (gs0, gs1)
    st_s = (ss0, ss1)

    wid = lax.axis_index("s") * NC + lax.axis_index("c")
    iota = lax.iota(jnp.int32, 16)

    def tile_qq(g):
        t = TPW * wid + g // 2
        return t, lax.rem(g, 2)

    def start_idx(g, b):
        t, qq = tile_qq(g)
        pltpu.async_copy(xr.at[t, pl.ds(4 * qq, 4)], idx_s[b], i_s[b])

    def wait_idx(b):
        pltpu.make_async_copy(
            xr.at[0, pl.ds(0, 4)], idx_s[b], i_s[b]
        ).wait()

    def fire(b):
        for d in range(4):
            pltpu.async_copy(
                table_hbm.at[idx_s[b].at[d]],
                rows_v[b].at[pl.ds(LANE * d, LANE)],
                gat_s[b],
            )

    def drain_gat(b):
        pltpu.make_async_copy(
            table_hbm.at[pl.ds(0, 4 * LANE)],
            rows_v[b],
            gat_s[b],
        ).wait()

    def out_slice(g):
        t, qq = tile_qq(g)
        h8 = t // CB
        c = lax.rem(t, CB)
        return out_hbm.at[pl.ds(32 * h8 + 16 * qq, 16), c]

    def extract(b):
        rows = rows_v[b]
        dst = dst_v[b]
        cols = [jnp.full((16,), e, jnp.int32) for e in range(EMBED_DIM)]
        for d in range(4):

            def jbody(j, carry, d=d):
                rowv = 128 * d + 16 * j + iota
                for half in range(2):
                    vs = [
                        plsc.load_gather(rows, [rowv, cols[16 * half + i]])
                        for i in range(16)
                    ]
                    for i in range(16):
                        e = 16 * half + i
                        dst[4 * d + e // 8, e % 8, pl.ds(16 * j, 16)] = vs[i]
                return carry

            lax.fori_loop(0, 8, jbody, 0)

    def start_store(g, b):
        pltpu.async_copy(dst_v[b], out_slice(g), st_s[b])

    def wait_store(g, b):
        pltpu.make_async_copy(dst_v[b], out_slice(g), st_s[b]).wait()

    def chunk(g, k, fire_next, wait_st, idx2):
        b = k % 2
        nb = (k + 1) % 2
        if fire_next:
            wait_idx(nb)
        drain_gat(b)
        if fire_next:
            fire(nb)
        if wait_st:
            wait_store(g - 2, b)
        extract(b)
        start_store(g, b)
        if idx2:
            start_idx(g + 2, b)

    start_idx(0, 0)
    start_idx(1, 1)
    wait_idx(0)
    fire(0)
    chunk(0, 0, True, False, True)
    chunk(1, 1, True, False, True)

    def step(it, carry):
        g = 2 + 2 * it
        chunk(g, 0, True, True, True)
        chunk(g + 1, 1, True, True, True)
        return carry

    lax.fori_loop(0, (CPW - 4) // 2, step, 0)

    chunk(CPW - 2, 0, True, True, False)
    chunk(CPW - 1, 1, False, True, False)
    wait_store(CPW - 2, 0)
    wait_store(CPW - 1, 1)


@jax.jit
def _embed(xr, table):
    mesh = plsc.VectorSubcoreMesh(core_axis_name="c", subcore_axis_name="s")
    fn = pl.kernel(
        _body,
        out_type=jax.ShapeDtypeStruct((4 * HIST, CB, 8, LANE), jnp.float32),
        mesh=mesh,
        scratch_types=[
            pltpu.VMEM((4, LANE), jnp.int32),
            pltpu.VMEM((4, LANE), jnp.int32),
            pltpu.VMEM((4 * LANE, EMBED_DIM), jnp.float32),
            pltpu.VMEM((4 * LANE, EMBED_DIM), jnp.float32),
            pltpu.VMEM((16, 8, LANE), jnp.float32),
            pltpu.VMEM((16, 8, LANE), jnp.float32),
            pltpu.SemaphoreType.DMA,
            pltpu.SemaphoreType.DMA,
            pltpu.SemaphoreType.DMA,
            pltpu.SemaphoreType.DMA,
            pltpu.SemaphoreType.DMA,
            pltpu.SemaphoreType.DMA,
        ],
        compiler_params=pltpu.CompilerParams(
            use_tc_tiling_on_sc=False, needs_layout_passes=False
        ),
    )
    return fn(xr, table)


def kernel(x, table):
    xi = x.astype(jnp.int32)
    xr = (
        xi.T.reshape(H8, 8, CB, LANE)
        .transpose(0, 2, 1, 3)
        .reshape(TILES, 8, LANE)
    )
    out5 = _embed(xr, table)
    out = (
        out5.reshape(HIST, 4, CB, 8, LANE)
        .transpose(2, 4, 0, 1, 3)
        .reshape(BATCH, HIST, EMBED_DIM)
    )
    return out


# speedup vs baseline: 1.3007x; 1.0014x over previous
import jax
import jax.numpy as jnp
from jax import lax
from jax.experimental import pallas as pl
from jax.experimental.pallas import tpu as pltpu
from jax.experimental.pallas import tpu_sc as plsc

VOCAB = 1000000
EMBED_DIM = 32
BATCH = 4096
HIST = 200

NC = 2
NS = 16
NW = NC * NS

LANE = 128
H8 = HIST // 8
CB = BATCH // LANE
TILES = H8 * CB
TPW = TILES // NW
CPW = TPW * 2


def _body(xr, table_hbm, out_hbm,
          idxs0, idxs1, rows0, rows1, dst0, dst1,
          is0, is1, gs0, gs1, ss0, ss1):
    idx_s = (idxs0, idxs1)
    rows_v = (rows0, rows1)
    dst_v = (dst0, dst1)
    i_s = (is0, is1)
    gat_s = (gs0, gs1)
    st_s = (ss0, ss1)

    wid = lax.axis_index("s") * NC + lax.axis_index("c")
    iota = lax.iota(jnp.int32, 16)

    def tile_qq(g):
        t = TPW * wid + g // 2
        return t, lax.rem(g, 2)

    def start_idx(g, b):
        t, qq = tile_qq(g)
        pltpu.async_copy(xr.at[t, pl.ds(4 * qq, 4)], idx_s[b], i_s[b])

    def wait_idx(b):
        pltpu.make_async_copy(
            xr.at[0, pl.ds(0, 4)], idx_s[b], i_s[b]
        ).wait()

    def fire(b):
        for d in range(4):
            pltpu.async_copy(
                table_hbm.at[idx_s[b].at[d]],
                rows_v[b].at[pl.ds(LANE * d, LANE)],
                gat_s[b],
            )

    def drain_gat(b):
        pltpu.make_async_copy(
            table_hbm.at[pl.ds(0, 4 * LANE)],
            rows_v[b],
            gat_s[b],
        ).wait()

    def out_slice(g):
        t, qq = tile_qq(g)
        h8 = t // CB
        c = lax.rem(t, CB)
        return out_hbm.at[pl.ds(32 * h8 + 16 * qq, 16), c]

    def extract(b):
        rows = rows_v[b]
        dst = dst_v[b]
        cols = [jnp.full((16,), e, jnp.int32) for e in range(EMBED_DIM)]
        for d in range(4):

            def jbody(jt, carry, d=d):
                for dj in range(2):
                    j = 2 * jt + dj
                    rowv = 128 * d + 16 * j + iota
                    for half in range(2):
                        vs = [
                            plsc.load_gather(rows, [rowv, cols[16 * half + i]])
                            for i in range(16)
                        ]
                        for i in range(16):
                            e = 16 * half + i
                            dst[4 * d + e // 8, e % 8, pl.ds(16 * j, 16)] = vs[i]
                return carry

            lax.fori_loop(0, 4, jbody, 0)

    def start_store(g, b):
        pltpu.async_copy(dst_v[b], out_slice(g), st_s[b])

    def wait_store(g, b):
        pltpu.make_async_copy(dst_v[b], out_slice(g), st_s[b]).wait()

    def chunk(g, k, fire_next, wait_st, idx2):
        b = k % 2
        nb = (k + 1) % 2
        if fire_next:
            wait_idx(nb)
        drain_gat(b)
        if fire_next:
            fire(nb)
        if wait_st:
            wait_store(g - 2, b)
        extract(b)
        start_store(g, b)
        if idx2:
            start_idx(g + 2, b)

    start_idx(0, 0)
    start_idx(1, 1)
    wait_idx(0)
    fire(0)
    chunk(0, 0, True, False, True)
    chunk(1, 1, True, False, True)

    def step(it, carry):
        g = 2 + 2 * it
        chunk(g, 0, True, True, True)
        chunk(g + 1, 1, True, True, True)
        return carry

    lax.fori_loop(0, (CPW - 4) // 2, step, 0)

    chunk(CPW - 2, 0, True, True, False)
    chunk(CPW - 1, 1, False, True, False)
    wait_store(CPW - 2, 0)
    wait_store(CPW - 1, 1)


@jax.jit
def _embed(xr, table):
    mesh = plsc.VectorSubcoreMesh(core_axis_name="c", subcore_axis_name="s")
    fn = pl.kernel(
        _body,
        out_type=jax.ShapeDtypeStruct((4 * HIST, CB, 8, LANE), jnp.float32),
        mesh=mesh,
        scratch_types=[
            pltpu.VMEM((4, LANE), jnp.int32),
            pltpu.VMEM((4, LANE), jnp.int32),
            pltpu.VMEM((4 * LANE, EMBED_DIM), jnp.float32),
            pltpu.VMEM((4 * LANE, EMBED_DIM), jnp.float32),
            pltpu.VMEM((16, 8, LANE), jnp.float32),
            pltpu.VMEM((16, 8, LANE), jnp.float32),
            pltpu.SemaphoreType.DMA,
            pltpu.SemaphoreType.DMA,
            pltpu.SemaphoreType.DMA,
            pltpu.SemaphoreType.DMA,
            pltpu.SemaphoreType.DMA,
            pltpu.SemaphoreType.DMA,
        ],
        compiler_params=pltpu.CompilerParams(
            use_tc_tiling_on_sc=False, needs_layout_passes=False
        ),
    )
    return fn(xr, table)


def kernel(x, table):
    xi = x.astype(jnp.int32)
    xr = (
        xi.T.reshape(H8, 8, CB, LANE)
        .transpose(0, 2, 1, 3)
        .reshape(TILES, 8, LANE)
    )
    out5 = _embed(xr, table)
    out = (
        out5.reshape(HIST, 4, CB, 8, LANE)
        .transpose(2, 4, 0, 1, 3)
        .reshape(BATCH, HIST, EMBED_DIM)
    )
    return out
